# Initial kernel scaffold; baseline (speedup 1.0000x reference)
import functools

import jax
import jax.numpy as jnp
from jax import lax
from jax.experimental import pallas as pl
from jax.experimental.pallas import tpu as pltpu
from jax.experimental.pallas import tpu_sc as plsc

N = 10000
E = 320000
B = 80
NB = E // B
NWORK = 32
NB_W = NB // NWORK
TILE_ROWS = N // 16
RB = 500


def _make_edge_kernel(dp):
  fg = dp // 16
  mesh = plsc.VectorSubcoreMesh(core_axis_name="c", subcore_axis_name="s")

  @functools.partial(
      pl.kernel,
      out_type=jax.ShapeDtypeStruct((2, N, dp), jnp.float32),
      mesh=mesh,
      scratch_types=[
          pltpu.VMEM((N,), jnp.float32),
          pltpu.VMEM((N,), jnp.float32),
          pltpu.VMEM((NB_W, B), jnp.int32),
          pltpu.VMEM((NB_W, B), jnp.int32),
          pltpu.VMEM((B,), jnp.float32),
          pltpu.VMEM((B, dp), jnp.float32),
          pltpu.VMEM((NB_W, dp), jnp.float32),
          pltpu.VMEM_SHARED((N, dp), jnp.float32),
          pltpu.SemaphoreType.DMA,
      ],
  )
  def edge_kernel(h_hbm, asrc_hbm, adst_hbm, srcb_hbm, dstb_hbm, out_hbm,
                  asrc_v, adst_v, srcb_v, dstb_v, ex_v, rows_v, zbuf,
                  acc_sh, sem):
    cid = lax.axis_index("c")
    sid = lax.axis_index("s")
    wid = cid * 16 + sid

    pltpu.sync_copy(asrc_hbm, asrc_v)
    pltpu.sync_copy(adst_hbm, adst_v)
    pltpu.sync_copy(srcb_hbm.at[pl.ds(wid * NB_W, NB_W)], srcb_v)
    pltpu.sync_copy(dstb_hbm.at[pl.ds(wid * NB_W, NB_W)], dstb_v)

    zeros16 = jnp.zeros((16,), jnp.float32)

    def zrow(i, carry):
      for f in range(fg):
        zbuf[i, pl.ds(f * 16, 16)] = zeros16
      return carry

    lax.fori_loop(0, NB_W, zrow, 0)
    for kk in range(TILE_ROWS // NB_W):
      pltpu.sync_copy(
          zbuf, acc_sh.at[pl.ds(sid * TILE_ROWS + kk * NB_W, NB_W)])
    plsc.subcore_barrier()

    def batch_body(j, carry):
      src_row = srcb_v.at[j]
      dst_row = dstb_v.at[j]
      gather = pltpu.async_copy(h_hbm.at[src_row], rows_v, sem)
      for g in range(B // 16):
        s16 = srcb_v[j, pl.ds(g * 16, 16)]
        d16 = dstb_v[j, pl.ds(g * 16, 16)]
        a_s = plsc.load_gather(asrc_v, [s16])
        a_d = plsc.load_gather(adst_v, [d16])
        e = a_s + a_d
        e = jnp.where(e < 0.0, e * jnp.float32(0.2), e)
        ex_v[pl.ds(g * 16, 16)] = jnp.exp(e)
      gather.wait()

      def scale_body(k, c2):
        exk = ex_v[k]
        for f in range(fg):
          rows_v[k, pl.ds(f * 16, 16)] = rows_v[k, pl.ds(f * 16, 16)] * exk
        return c2

      lax.fori_loop(0, B, scale_body, 0)
      pltpu.sync_copy(rows_v, acc_sh.at[dst_row], add=True)
      return carry

    lax.fori_loop(0, NB_W, batch_body, 0)
    plsc.subcore_barrier()

    pltpu.sync_copy(acc_sh.at[pl.ds(sid * TILE_ROWS, TILE_ROWS)],
                    out_hbm.at[cid, pl.ds(sid * TILE_ROWS, TILE_ROWS)])

  return edge_kernel


_edge128 = _make_edge_kernel(144)
_edge64 = _make_edge_kernel(80)


def _dense_first_body(x_ref, w_ref, as_ref, ad_ref, h_ref, asrc_ref, adst_ref,
                      *, d_out, dp_out):
  h = jnp.dot(x_ref[...], w_ref[...], preferred_element_type=jnp.float32)
  h_ref[:, :d_out] = h
  col = lax.broadcasted_iota(jnp.int32, (RB, dp_out - d_out), 1)
  h_ref[:, d_out:] = jnp.where(col == 0, jnp.float32(1.0), jnp.float32(0.0))
  asrc_ref[...] = jnp.dot(h, as_ref[...], preferred_element_type=jnp.float32)
  adst_ref[...] = jnp.dot(h, ad_ref[...], preferred_element_type=jnp.float32)


def _dense_mid_body(acc_ref, b_ref, w_ref, as_ref, ad_ref,
                    h_ref, asrc_ref, adst_ref, *, d_in, d_out, dp_out):
  a0 = acc_ref[0]
  a1 = acc_ref[1]
  num = a0[:, :d_in] + a1[:, :d_in]
  den = a0[:, d_in:d_in + 1] + a1[:, d_in:d_in + 1]
  hprev = jnp.maximum(num / (den + jnp.float32(1e-16)) + b_ref[...], 0.0)
  h = jnp.dot(hprev, w_ref[...], preferred_element_type=jnp.float32)
  h_ref[:, :d_out] = h
  col = lax.broadcasted_iota(jnp.int32, (RB, dp_out - d_out), 1)
  h_ref[:, d_out:] = jnp.where(col == 0, jnp.float32(1.0), jnp.float32(0.0))
  asrc_ref[...] = jnp.dot(h, as_ref[...], preferred_element_type=jnp.float32)
  adst_ref[...] = jnp.dot(h, ad_ref[...], preferred_element_type=jnp.float32)


def _final_body(acc_ref, b_ref, o_ref, *, d):
  a0 = acc_ref[0]
  a1 = acc_ref[1]
  num = a0[:, :d] + a1[:, :d]
  den = a0[:, d:d + 1] + a1[:, d:d + 1]
  h = jnp.maximum(num / (den + jnp.float32(1e-16)) + b_ref[...], 0.0)
  m = jnp.max(h, axis=-1, keepdims=True)
  ex = jnp.exp(h - m)
  o_ref[...] = h - m - jnp.log(jnp.sum(ex, axis=-1, keepdims=True))


def _dense_first(x, w, a_s, a_d, d_out, dp_out):
  grid = (N // RB,)
  return pl.pallas_call(
      functools.partial(_dense_first_body, d_out=d_out, dp_out=dp_out),
      grid=grid,
      in_specs=[
          pl.BlockSpec((RB, x.shape[1]), lambda i: (i, 0)),
          pl.BlockSpec(w.shape, lambda i: (0, 0)),
          pl.BlockSpec((d_out, 1), lambda i: (0, 0)),
          pl.BlockSpec((d_out, 1), lambda i: (0, 0)),
      ],
      out_specs=[
          pl.BlockSpec((RB, dp_out), lambda i: (i, 0)),
          pl.BlockSpec((RB, 1), lambda i: (i, 0)),
          pl.BlockSpec((RB, 1), lambda i: (i, 0)),
      ],
      out_shape=[
          jax.ShapeDtypeStruct((N, dp_out), jnp.float32),
          jax.ShapeDtypeStruct((N, 1), jnp.float32),
          jax.ShapeDtypeStruct((N, 1), jnp.float32),
      ],
  )(x, w, a_s, a_d)


def _dense_mid(acc, b, w, a_s, a_d, d_in, d_out, dp_out):
  grid = (N // RB,)
  dp_in = acc.shape[2]
  return pl.pallas_call(
      functools.partial(_dense_mid_body, d_in=d_in, d_out=d_out,
                        dp_out=dp_out),
      grid=grid,
      in_specs=[
          pl.BlockSpec((2, RB, dp_in), lambda i: (0, i, 0)),
          pl.BlockSpec((1, d_in), lambda i: (0, 0)),
          pl.BlockSpec(w.shape, lambda i: (0, 0)),
          pl.BlockSpec((d_out, 1), lambda i: (0, 0)),
          pl.BlockSpec((d_out, 1), lambda i: (0, 0)),
      ],
      out_specs=[
          pl.BlockSpec((RB, dp_out), lambda i: (i, 0)),
          pl.BlockSpec((RB, 1), lambda i: (i, 0)),
          pl.BlockSpec((RB, 1), lambda i: (i, 0)),
      ],
      out_shape=[
          jax.ShapeDtypeStruct((N, dp_out), jnp.float32),
          jax.ShapeDtypeStruct((N, 1), jnp.float32),
          jax.ShapeDtypeStruct((N, 1), jnp.float32),
      ],
  )(acc, b, w, a_s, a_d)


def _final(acc, b, d):
  grid = (N // RB,)
  dp_in = acc.shape[2]
  return pl.pallas_call(
      functools.partial(_final_body, d=d),
      grid=grid,
      in_specs=[
          pl.BlockSpec((2, RB, dp_in), lambda i: (0, i, 0)),
          pl.BlockSpec((1, d), lambda i: (0, 0)),
      ],
      out_specs=pl.BlockSpec((RB, d), lambda i: (i, 0)),
      out_shape=jax.ShapeDtypeStruct((N, d), jnp.float32),
  )(acc, b)


def kernel(x, edge_index, edge_attr, W1, a_src1, a_dst1, b1,
           W2, a_src2, a_dst2, b2, W3, a_src3, a_dst3, b3):
  del edge_attr
  srcb = edge_index[0].reshape(NB, B)
  dstb = edge_index[1].reshape(NB, B)

  h1, as1, ad1 = _dense_first(x, W1, a_src1.reshape(-1, 1),
                              a_dst1.reshape(-1, 1), 128, 144)
  acc1 = _edge128(h1, as1.reshape(-1), ad1.reshape(-1), srcb, dstb)

  h2, as2, ad2 = _dense_mid(acc1, b1.reshape(1, -1), W2,
                            a_src2.reshape(-1, 1), a_dst2.reshape(-1, 1),
                            128, 128, 144)
  acc2 = _edge128(h2, as2.reshape(-1), ad2.reshape(-1), srcb, dstb)

  h3, as3, ad3 = _dense_mid(acc2, b2.reshape(1, -1), W3,
                            a_src3.reshape(-1, 1), a_dst3.reshape(-1, 1),
                            128, 64, 80)
  acc3 = _edge64(h3, as3.reshape(-1), ad3.reshape(-1), srcb, dstb)

  return _final(acc3, b3.reshape(1, -1), 64)


# speedup vs baseline: 24.0775x; 24.0775x over previous
import functools

import jax
import jax.numpy as jnp
from jax import lax
from jax.experimental import pallas as pl
from jax.experimental.pallas import tpu as pltpu
from jax.experimental.pallas import tpu_sc as plsc

N = 10000
E = 320000
B = 80
NB = E // B
NBT = NB // 16
TILE_ROWS = 624
TAIL_ROWS = N - 16 * TILE_ROWS
RB = 400


def _make_edge_kernel(dp):
  fg = dp // 16
  mesh = plsc.VectorSubcoreMesh(core_axis_name="c", subcore_axis_name="s")

  @functools.partial(
      pl.kernel,
      out_type=jax.ShapeDtypeStruct((2, N, dp), jnp.float32),
      mesh=mesh,
      compiler_params=pltpu.CompilerParams(
          needs_layout_passes=False, use_tc_tiling_on_sc=False),
      scratch_types=[
          pltpu.VMEM((N,), jnp.float32),
          pltpu.VMEM((N,), jnp.float32),
          pltpu.VMEM((NBT, B), jnp.int32),
          pltpu.VMEM((NBT, B), jnp.int32),
          pltpu.VMEM((B,), jnp.float32),
          pltpu.VMEM((B, dp), jnp.float32),
          pltpu.VMEM_SHARED((N, dp), jnp.float32),
          pltpu.SemaphoreType.DMA,
      ],
  )
  def edge_kernel(h_hbm, asrc_hbm, adst_hbm, srcb_hbm, dstb_hbm, out_hbm,
                  asrc_v, adst_v, srcb_v, dstb_v, ex_v, rows_v,
                  acc_sh, sem):
    cid = lax.axis_index("c")
    sid = lax.axis_index("s")

    pltpu.sync_copy(asrc_hbm, asrc_v)
    pltpu.sync_copy(adst_hbm, adst_v)
    pltpu.sync_copy(srcb_hbm.at[sid], srcb_v)
    pltpu.sync_copy(dstb_hbm.at[sid], dstb_v)

    zeros16 = jnp.zeros((16,), jnp.float32)

    def zrow(i, carry):
      for f in range(fg):
        rows_v[i, pl.ds(f * 16, 16)] = zeros16
      return carry

    lax.fori_loop(0, B, zrow, 0)
    for kk in range(7):
      pltpu.sync_copy(rows_v, acc_sh.at[pl.ds(sid * TILE_ROWS + kk * 80, 80)])
    pltpu.sync_copy(rows_v.at[pl.ds(0, 64)],
                    acc_sh.at[pl.ds(sid * TILE_ROWS + 560, 64)])

    @pl.when(sid == 15)
    def _zero_tail():
      pltpu.sync_copy(rows_v.at[pl.ds(0, TAIL_ROWS)],
                      acc_sh.at[pl.ds(16 * TILE_ROWS, TAIL_ROWS)])

    plsc.subcore_barrier()

    table = h_hbm.at[cid]

    def batch_body(j, carry):
      src_row = srcb_v.at[j]
      dst_row = dstb_v.at[j]
      gather = pltpu.async_copy(table.at[src_row], rows_v, sem)
      for g in range(B // 16):
        s16 = srcb_v[j, pl.ds(g * 16, 16)]
        d16 = dstb_v[j, pl.ds(g * 16, 16)]
        a_s = plsc.load_gather(asrc_v, [s16])
        a_d = plsc.load_gather(adst_v, [d16])
        e = a_s + a_d
        e = jnp.where(e < 0.0, e * jnp.float32(0.2), e)
        ex_v[pl.ds(g * 16, 16)] = jnp.exp(e)
      gather.wait()

      def scale_group(g, c2):
        ex16 = ex_v[pl.ds(g * 16, 16)]
        for k16 in range(16):
          exk = ex16[k16]
          row = g * 16 + k16
          for f in range(fg):
            rows_v[row, pl.ds(f * 16, 16)] = (
                rows_v[row, pl.ds(f * 16, 16)] * exk)
        return c2

      lax.fori_loop(0, B // 16, scale_group, 0)
      pltpu.sync_copy(rows_v, acc_sh.at[dst_row], add=True)
      return carry

    lax.fori_loop(0, NBT, batch_body, 0)
    plsc.subcore_barrier()

    pltpu.sync_copy(acc_sh.at[pl.ds(sid * TILE_ROWS, TILE_ROWS)],
                    out_hbm.at[cid, pl.ds(sid * TILE_ROWS, TILE_ROWS)])

    @pl.when(sid == 15)
    def _copy_tail():
      pltpu.sync_copy(acc_sh.at[pl.ds(16 * TILE_ROWS, TAIL_ROWS)],
                      out_hbm.at[cid, pl.ds(16 * TILE_ROWS, TAIL_ROWS)])

  return edge_kernel


_edge80 = _make_edge_kernel(80)
_edge48 = _make_edge_kernel(48)


def _write_halves(h_ref, h, d_out, dp_out):
  half = d_out // 2
  pad = dp_out - half
  col = lax.broadcasted_iota(jnp.int32, (RB, pad), 1)
  ind = jnp.where(col == 0, jnp.float32(1.0), jnp.float32(0.0))
  h_ref[0, :, :half] = h[:, :half]
  h_ref[0, :, half:] = ind
  h_ref[1, :, :half] = h[:, half:]
  h_ref[1, :, half:] = ind


def _combine(acc_ref, b_ref, d_in):
  half = d_in // 2
  num = jnp.concatenate([acc_ref[0, :, :half], acc_ref[1, :, :half]], axis=1)
  den = acc_ref[0, :, half:half + 1]
  return jnp.maximum(num / (den + jnp.float32(1e-16)) + b_ref[...], 0.0)


def _dense_first_body(x_ref, w_ref, as_ref, ad_ref, h_ref, asrc_ref, adst_ref,
                      *, d_out, dp_out):
  h = jnp.dot(x_ref[...], w_ref[...], preferred_element_type=jnp.float32)
  _write_halves(h_ref, h, d_out, dp_out)
  asrc_ref[...] = jnp.dot(h, as_ref[...], preferred_element_type=jnp.float32)
  adst_ref[...] = jnp.dot(h, ad_ref[...], preferred_element_type=jnp.float32)


def _dense_mid_body(acc_ref, b_ref, w_ref, as_ref, ad_ref,
                    h_ref, asrc_ref, adst_ref, *, d_in, d_out, dp_out):
  hprev = _combine(acc_ref, b_ref, d_in)
  h = jnp.dot(hprev, w_ref[...], preferred_element_type=jnp.float32)
  _write_halves(h_ref, h, d_out, dp_out)
  asrc_ref[...] = jnp.dot(h, as_ref[...], preferred_element_type=jnp.float32)
  adst_ref[...] = jnp.dot(h, ad_ref[...], preferred_element_type=jnp.float32)


def _final_body(acc_ref, b_ref, o_ref, *, d):
  h = _combine(acc_ref, b_ref, d)
  m = jnp.max(h, axis=-1, keepdims=True)
  ex = jnp.exp(h - m)
  o_ref[...] = h - m - jnp.log(jnp.sum(ex, axis=-1, keepdims=True))


def _dense_first(x, w, a_s, a_d, d_out, dp_out):
  return pl.pallas_call(
      functools.partial(_dense_first_body, d_out=d_out, dp_out=dp_out),
      grid=(N // RB,),
      in_specs=[
          pl.BlockSpec((RB, x.shape[1]), lambda i: (i, 0)),
          pl.BlockSpec(w.shape, lambda i: (0, 0)),
          pl.BlockSpec((d_out, 1), lambda i: (0, 0)),
          pl.BlockSpec((d_out, 1), lambda i: (0, 0)),
      ],
      out_specs=[
          pl.BlockSpec((2, RB, dp_out), lambda i: (0, i, 0)),
          pl.BlockSpec((RB, 1), lambda i: (i, 0)),
          pl.BlockSpec((RB, 1), lambda i: (i, 0)),
      ],
      out_shape=[
          jax.ShapeDtypeStruct((2, N, dp_out), jnp.float32),
          jax.ShapeDtypeStruct((N, 1), jnp.float32),
          jax.ShapeDtypeStruct((N, 1), jnp.float32),
      ],
  )(x, w, a_s, a_d)


def _dense_mid(acc, b, w, a_s, a_d, d_in, d_out, dp_out):
  dp_in = acc.shape[2]
  return pl.pallas_call(
      functools.partial(_dense_mid_body, d_in=d_in, d_out=d_out,
                        dp_out=dp_out),
      grid=(N // RB,),
      in_specs=[
          pl.BlockSpec((2, RB, dp_in), lambda i: (0, i, 0)),
          pl.BlockSpec((1, d_in), lambda i: (0, 0)),
          pl.BlockSpec(w.shape, lambda i: (0, 0)),
          pl.BlockSpec((d_out, 1), lambda i: (0, 0)),
          pl.BlockSpec((d_out, 1), lambda i: (0, 0)),
      ],
      out_specs=[
          pl.BlockSpec((2, RB, dp_out), lambda i: (0, i, 0)),
          pl.BlockSpec((RB, 1), lambda i: (i, 0)),
          pl.BlockSpec((RB, 1), lambda i: (i, 0)),
      ],
      out_shape=[
          jax.ShapeDtypeStruct((2, N, dp_out), jnp.float32),
          jax.ShapeDtypeStruct((N, 1), jnp.float32),
          jax.ShapeDtypeStruct((N, 1), jnp.float32),
      ],
  )(acc, b, w, a_s, a_d)


def _final(acc, b, d):
  dp_in = acc.shape[2]
  return pl.pallas_call(
      functools.partial(_final_body, d=d),
      grid=(N // RB,),
      in_specs=[
          pl.BlockSpec((2, RB, dp_in), lambda i: (0, i, 0)),
          pl.BlockSpec((1, d), lambda i: (0, 0)),
      ],
      out_specs=pl.BlockSpec((RB, d), lambda i: (i, 0)),
      out_shape=jax.ShapeDtypeStruct((N, d), jnp.float32),
  )(acc, b)


def kernel(x, edge_index, edge_attr, W1, a_src1, a_dst1, b1,
           W2, a_src2, a_dst2, b2, W3, a_src3, a_dst3, b3):
  del edge_attr
  srcb = edge_index[0].reshape(16, NBT, B)
  dstb = edge_index[1].reshape(16, NBT, B)

  h1, as1, ad1 = _dense_first(x, W1, a_src1.reshape(-1, 1),
                              a_dst1.reshape(-1, 1), 128, 80)
  acc1 = _edge80(h1, as1.reshape(-1), ad1.reshape(-1), srcb, dstb)

  h2, as2, ad2 = _dense_mid(acc1, b1.reshape(1, -1), W2,
                            a_src2.reshape(-1, 1), a_dst2.reshape(-1, 1),
                            128, 128, 80)
  acc2 = _edge80(h2, as2.reshape(-1), ad2.reshape(-1), srcb, dstb)

  h3, as3, ad3 = _dense_mid(acc2, b2.reshape(1, -1), W3,
                            a_src3.reshape(-1, 1), a_dst3.reshape(-1, 1),
                            128, 64, 48)
  acc3 = _edge48(h3, as3.reshape(-1), ad3.reshape(-1), srcb, dstb)

  return _final(acc3, b3.reshape(1, -1), 64)


# speedup vs baseline: 34.1065x; 1.4165x over previous
import functools

import jax
import jax.numpy as jnp
from jax import lax
from jax.experimental import pallas as pl
from jax.experimental.pallas import tpu as pltpu
from jax.experimental.pallas import tpu_sc as plsc

N = 10000
E = 320000
B = 80
NB = E // B
NBT = NB // 16
TILE_ROWS = 624
TAIL_ROWS = N - 16 * TILE_ROWS
RB = 400


def _make_edge_kernel(dp):
  fg = dp // 16
  mesh = plsc.VectorSubcoreMesh(core_axis_name="c", subcore_axis_name="s")

  @functools.partial(
      pl.kernel,
      out_type=jax.ShapeDtypeStruct((2, N, dp), jnp.float32),
      mesh=mesh,
      compiler_params=pltpu.CompilerParams(
          needs_layout_passes=False, use_tc_tiling_on_sc=False),
      scratch_types=[
          pltpu.VMEM((N,), jnp.float32),
          pltpu.VMEM((N,), jnp.float32),
          pltpu.VMEM((NBT, B), jnp.int32),
          pltpu.VMEM((NBT, B), jnp.int32),
          pltpu.VMEM((B,), jnp.float32),
          pltpu.VMEM((2, B, dp), jnp.float32),
          pltpu.VMEM_SHARED((N, dp), jnp.float32),
          pltpu.SemaphoreType.DMA,
          pltpu.SemaphoreType.DMA,
      ],
  )
  def edge_kernel(h_hbm, asrc_hbm, adst_hbm, srcb_hbm, dstb_hbm, out_hbm,
                  asrc_v, adst_v, srcb_v, dstb_v, ex_v, rows2_v,
                  acc_sh, gsem, ssem):
    cid = lax.axis_index("c")
    sid = lax.axis_index("s")

    pltpu.sync_copy(asrc_hbm, asrc_v)
    pltpu.sync_copy(adst_hbm, adst_v)
    pltpu.sync_copy(srcb_hbm.at[sid], srcb_v)
    pltpu.sync_copy(dstb_hbm.at[sid], dstb_v)

    zeros16 = jnp.zeros((16,), jnp.float32)
    zrows = rows2_v.at[0]

    def zrow(i, carry):
      for f in range(fg):
        zrows[i, pl.ds(f * 16, 16)] = zeros16
      return carry

    lax.fori_loop(0, B, zrow, 0)
    for kk in range(7):
      pltpu.sync_copy(zrows, acc_sh.at[pl.ds(sid * TILE_ROWS + kk * 80, 80)])
    pltpu.sync_copy(zrows.at[pl.ds(0, 64)],
                    acc_sh.at[pl.ds(sid * TILE_ROWS + 560, 64)])

    @pl.when(sid == 15)
    def _zero_tail():
      pltpu.sync_copy(zrows.at[pl.ds(0, TAIL_ROWS)],
                      acc_sh.at[pl.ds(16 * TILE_ROWS, TAIL_ROWS)])

    plsc.subcore_barrier()

    table = h_hbm.at[cid]

    pltpu.async_copy(table.at[srcb_v.at[0]], rows2_v.at[0], gsem)

    def do_batch(j, buf, other, first, last):
      rows_v = rows2_v.at[buf]

      @pl.when(jnp.logical_not(first))
      def _drain_scatter():
        pltpu.make_async_copy(
            rows2_v.at[other], acc_sh.at[pl.ds(0, B)], ssem).wait()

      @pl.when(jnp.logical_not(last))
      def _issue_gather():
        pltpu.async_copy(table.at[srcb_v.at[j + 1]], rows2_v.at[other], gsem)

      for g in range(B // 16):
        s16 = srcb_v[j, pl.ds(g * 16, 16)]
        d16 = dstb_v[j, pl.ds(g * 16, 16)]
        a_s = plsc.load_gather(asrc_v, [s16])
        a_d = plsc.load_gather(adst_v, [d16])
        e = a_s + a_d
        e = jnp.where(e < 0.0, e * jnp.float32(0.2), e)
        ex_v[pl.ds(g * 16, 16)] = jnp.exp(e)

      pltpu.make_async_copy(table.at[srcb_v.at[j]], rows_v, gsem).wait()

      def scale_group(g, c2):
        ex16 = ex_v[pl.ds(g * 16, 16)]
        for k16 in range(16):
          exk = ex16[k16]
          row = g * 16 + k16
          for f in range(fg):
            rows_v[row, pl.ds(f * 16, 16)] = (
                rows_v[row, pl.ds(f * 16, 16)] * exk)
        return c2

      lax.fori_loop(0, B // 16, scale_group, 0)
      pltpu.async_copy(rows_v, acc_sh.at[dstb_v.at[j]], ssem, add=True)

    def batch_pair(j2, carry):
      do_batch(2 * j2, 0, 1, first=j2 == 0, last=jnp.bool_(False))
      do_batch(2 * j2 + 1, 1, 0, first=jnp.bool_(False),
               last=j2 == NBT // 2 - 1)
      return carry

    lax.fori_loop(0, NBT // 2, batch_pair, 0)
    pltpu.make_async_copy(rows2_v.at[1], acc_sh.at[pl.ds(0, B)], ssem).wait()
    plsc.subcore_barrier()

    pltpu.sync_copy(acc_sh.at[pl.ds(sid * TILE_ROWS, TILE_ROWS)],
                    out_hbm.at[cid, pl.ds(sid * TILE_ROWS, TILE_ROWS)])

    @pl.when(sid == 15)
    def _copy_tail():
      pltpu.sync_copy(acc_sh.at[pl.ds(16 * TILE_ROWS, TAIL_ROWS)],
                      out_hbm.at[cid, pl.ds(16 * TILE_ROWS, TAIL_ROWS)])

  return edge_kernel


_edge80 = _make_edge_kernel(80)
_edge48 = _make_edge_kernel(48)


def _write_halves(h_ref, h, d_out, dp_out):
  half = d_out // 2
  pad = dp_out - half
  col = lax.broadcasted_iota(jnp.int32, (RB, pad), 1)
  ind = jnp.where(col == 0, jnp.float32(1.0), jnp.float32(0.0))
  h_ref[0, :, :half] = h[:, :half]
  h_ref[0, :, half:] = ind
  h_ref[1, :, :half] = h[:, half:]
  h_ref[1, :, half:] = ind


def _combine(acc_ref, b_ref, d_in):
  half = d_in // 2
  num = jnp.concatenate([acc_ref[0, :, :half], acc_ref[1, :, :half]], axis=1)
  den = acc_ref[0, :, half:half + 1]
  return jnp.maximum(num / (den + jnp.float32(1e-16)) + b_ref[...], 0.0)


def _dense_first_body(x_ref, w_ref, as_ref, ad_ref, h_ref, asrc_ref, adst_ref,
                      *, d_out, dp_out):
  h = jnp.dot(x_ref[...], w_ref[...], preferred_element_type=jnp.float32)
  _write_halves(h_ref, h, d_out, dp_out)
  asrc_ref[...] = jnp.dot(h, as_ref[...], preferred_element_type=jnp.float32)
  adst_ref[...] = jnp.dot(h, ad_ref[...], preferred_element_type=jnp.float32)


def _dense_mid_body(acc_ref, b_ref, w_ref, as_ref, ad_ref,
                    h_ref, asrc_ref, adst_ref, *, d_in, d_out, dp_out):
  hprev = _combine(acc_ref, b_ref, d_in)
  h = jnp.dot(hprev, w_ref[...], preferred_element_type=jnp.float32)
  _write_halves(h_ref, h, d_out, dp_out)
  asrc_ref[...] = jnp.dot(h, as_ref[...], preferred_element_type=jnp.float32)
  adst_ref[...] = jnp.dot(h, ad_ref[...], preferred_element_type=jnp.float32)


def _final_body(acc_ref, b_ref, o_ref, *, d):
  h = _combine(acc_ref, b_ref, d)
  m = jnp.max(h, axis=-1, keepdims=True)
  ex = jnp.exp(h - m)
  o_ref[...] = h - m - jnp.log(jnp.sum(ex, axis=-1, keepdims=True))


def _dense_first(x, w, a_s, a_d, d_out, dp_out):
  return pl.pallas_call(
      functools.partial(_dense_first_body, d_out=d_out, dp_out=dp_out),
      grid=(N // RB,),
      in_specs=[
          pl.BlockSpec((RB, x.shape[1]), lambda i: (i, 0)),
          pl.BlockSpec(w.shape, lambda i: (0, 0)),
          pl.BlockSpec((d_out, 1), lambda i: (0, 0)),
          pl.BlockSpec((d_out, 1), lambda i: (0, 0)),
      ],
      out_specs=[
          pl.BlockSpec((2, RB, dp_out), lambda i: (0, i, 0)),
          pl.BlockSpec((RB, 1), lambda i: (i, 0)),
          pl.BlockSpec((RB, 1), lambda i: (i, 0)),
      ],
      out_shape=[
          jax.ShapeDtypeStruct((2, N, dp_out), jnp.float32),
          jax.ShapeDtypeStruct((N, 1), jnp.float32),
          jax.ShapeDtypeStruct((N, 1), jnp.float32),
      ],
  )(x, w, a_s, a_d)


def _dense_mid(acc, b, w, a_s, a_d, d_in, d_out, dp_out):
  dp_in = acc.shape[2]
  return pl.pallas_call(
      functools.partial(_dense_mid_body, d_in=d_in, d_out=d_out,
                        dp_out=dp_out),
      grid=(N // RB,),
      in_specs=[
          pl.BlockSpec((2, RB, dp_in), lambda i: (0, i, 0)),
          pl.BlockSpec((1, d_in), lambda i: (0, 0)),
          pl.BlockSpec(w.shape, lambda i: (0, 0)),
          pl.BlockSpec((d_out, 1), lambda i: (0, 0)),
          pl.BlockSpec((d_out, 1), lambda i: (0, 0)),
      ],
      out_specs=[
          pl.BlockSpec((2, RB, dp_out), lambda i: (0, i, 0)),
          pl.BlockSpec((RB, 1), lambda i: (i, 0)),
          pl.BlockSpec((RB, 1), lambda i: (i, 0)),
      ],
      out_shape=[
          jax.ShapeDtypeStruct((2, N, dp_out), jnp.float32),
          jax.ShapeDtypeStruct((N, 1), jnp.float32),
          jax.ShapeDtypeStruct((N, 1), jnp.float32),
      ],
  )(acc, b, w, a_s, a_d)


def _final(acc, b, d):
  dp_in = acc.shape[2]
  return pl.pallas_call(
      functools.partial(_final_body, d=d),
      grid=(N // RB,),
      in_specs=[
          pl.BlockSpec((2, RB, dp_in), lambda i: (0, i, 0)),
          pl.BlockSpec((1, d), lambda i: (0, 0)),
      ],
      out_specs=pl.BlockSpec((RB, d), lambda i: (i, 0)),
      out_shape=jax.ShapeDtypeStruct((N, d), jnp.float32),
  )(acc, b)


def kernel(x, edge_index, edge_attr, W1, a_src1, a_dst1, b1,
           W2, a_src2, a_dst2, b2, W3, a_src3, a_dst3, b3):
  del edge_attr
  srcb = edge_index[0].reshape(16, NBT, B)
  dstb = edge_index[1].reshape(16, NBT, B)

  h1, as1, ad1 = _dense_first(x, W1, a_src1.reshape(-1, 1),
                              a_dst1.reshape(-1, 1), 128, 80)
  acc1 = _edge80(h1, as1.reshape(-1), ad1.reshape(-1), srcb, dstb)

  h2, as2, ad2 = _dense_mid(acc1, b1.reshape(1, -1), W2,
                            a_src2.reshape(-1, 1), a_dst2.reshape(-1, 1),
                            128, 128, 80)
  acc2 = _edge80(h2, as2.reshape(-1), ad2.reshape(-1), srcb, dstb)

  h3, as3, ad3 = _dense_mid(acc2, b2.reshape(1, -1), W3,
                            a_src3.reshape(-1, 1), a_dst3.reshape(-1, 1),
                            128, 64, 48)
  acc3 = _edge48(h3, as3.reshape(-1), ad3.reshape(-1), srcb, dstb)

  return _final(acc3, b3.reshape(1, -1), 64)


# speedup vs baseline: 34.6422x; 1.0157x over previous
import functools

import jax
import jax.numpy as jnp
from jax import lax
from jax.experimental import pallas as pl
from jax.experimental.pallas import tpu as pltpu
from jax.experimental.pallas import tpu_sc as plsc

N = 10000
E = 320000
B = 80
NB = E // B
NBT = NB // 16
TILE_ROWS = 624
TAIL_ROWS = N - 16 * TILE_ROWS
RB = 400


def _make_edge_kernel(dp):
  fg = dp // 16
  mesh = plsc.VectorSubcoreMesh(core_axis_name="c", subcore_axis_name="s")

  @functools.partial(
      pl.kernel,
      out_type=jax.ShapeDtypeStruct((2, N, dp), jnp.float32),
      mesh=mesh,
      compiler_params=pltpu.CompilerParams(
          needs_layout_passes=False, use_tc_tiling_on_sc=False),
      scratch_types=[
          pltpu.VMEM((N,), jnp.float32),
          pltpu.VMEM((N,), jnp.float32),
          pltpu.VMEM((NBT, B), jnp.int32),
          pltpu.VMEM((NBT, B), jnp.int32),
          pltpu.VMEM((B,), jnp.float32),
          pltpu.VMEM((2, B, dp), jnp.float32),
          pltpu.VMEM_SHARED((N, dp), jnp.float32),
          pltpu.SemaphoreType.DMA,
          pltpu.SemaphoreType.DMA,
      ],
  )
  def edge_kernel(h_hbm, asrc_hbm, adst_hbm, srcb_hbm, dstb_hbm, out_hbm,
                  asrc_v, adst_v, srcb_v, dstb_v, ex_v, rows2_v,
                  acc_sh, gsem, ssem):
    cid = lax.axis_index("c")
    sid = lax.axis_index("s")

    pltpu.sync_copy(asrc_hbm, asrc_v)
    pltpu.sync_copy(adst_hbm, adst_v)
    pltpu.sync_copy(srcb_hbm.at[sid], srcb_v)
    pltpu.sync_copy(dstb_hbm.at[sid], dstb_v)

    zeros16 = jnp.zeros((16,), jnp.float32)
    zrows = rows2_v.at[0]

    def zrow(i, carry):
      for f in range(fg):
        zrows[i, pl.ds(f * 16, 16)] = zeros16
      return carry

    lax.fori_loop(0, B, zrow, 0)
    for kk in range(7):
      pltpu.sync_copy(zrows, acc_sh.at[pl.ds(sid * TILE_ROWS + kk * 80, 80)])
    pltpu.sync_copy(zrows.at[pl.ds(0, 64)],
                    acc_sh.at[pl.ds(sid * TILE_ROWS + 560, 64)])

    @pl.when(sid == 15)
    def _zero_tail():
      pltpu.sync_copy(zrows.at[pl.ds(0, TAIL_ROWS)],
                      acc_sh.at[pl.ds(16 * TILE_ROWS, TAIL_ROWS)])

    plsc.subcore_barrier()

    table = h_hbm.at[cid]

    pltpu.async_copy(table.at[srcb_v.at[0]], rows2_v.at[0], gsem)

    def do_batch(j, buf, other, first, last):
      rows_v = rows2_v.at[buf]

      @pl.when(jnp.logical_not(first))
      def _drain_scatter():
        pltpu.make_async_copy(
            rows2_v.at[other], acc_sh.at[pl.ds(0, B)], ssem).wait()

      @pl.when(jnp.logical_not(last))
      def _issue_gather():
        pltpu.async_copy(table.at[srcb_v.at[j + 1]], rows2_v.at[other], gsem)

      for g in range(B // 16):
        s16 = srcb_v[j, pl.ds(g * 16, 16)]
        d16 = dstb_v[j, pl.ds(g * 16, 16)]
        a_s = plsc.load_gather(asrc_v, [s16])
        a_d = plsc.load_gather(adst_v, [d16])
        e = a_s + a_d
        e = jnp.where(e < 0.0, e * jnp.float32(0.2), e)
        ex_v[pl.ds(g * 16, 16)] = jnp.exp(e)

      pltpu.make_async_copy(table.at[srcb_v.at[j]], rows_v, gsem).wait()

      for g in range(B // 16):
        ex16 = ex_v[pl.ds(g * 16, 16)]
        for k16 in range(16):
          exk = ex16[k16]
          row = g * 16 + k16
          for f in range(fg):
            rows_v[row, pl.ds(f * 16, 16)] = (
                rows_v[row, pl.ds(f * 16, 16)] * exk)
      pltpu.async_copy(rows_v, acc_sh.at[dstb_v.at[j]], ssem, add=True)

    def batch_pair(j2, carry):
      do_batch(2 * j2, 0, 1, first=j2 == 0, last=jnp.bool_(False))
      do_batch(2 * j2 + 1, 1, 0, first=jnp.bool_(False),
               last=j2 == NBT // 2 - 1)
      return carry

    lax.fori_loop(0, NBT // 2, batch_pair, 0)
    pltpu.make_async_copy(rows2_v.at[1], acc_sh.at[pl.ds(0, B)], ssem).wait()
    plsc.subcore_barrier()

    pltpu.sync_copy(acc_sh.at[pl.ds(sid * TILE_ROWS, TILE_ROWS)],
                    out_hbm.at[cid, pl.ds(sid * TILE_ROWS, TILE_ROWS)])

    @pl.when(sid == 15)
    def _copy_tail():
      pltpu.sync_copy(acc_sh.at[pl.ds(16 * TILE_ROWS, TAIL_ROWS)],
                      out_hbm.at[cid, pl.ds(16 * TILE_ROWS, TAIL_ROWS)])

  return edge_kernel


_edge80 = _make_edge_kernel(80)
_edge48 = _make_edge_kernel(48)


def _write_halves(h_ref, h, d_out, dp_out):
  half = d_out // 2
  pad = dp_out - half
  col = lax.broadcasted_iota(jnp.int32, (RB, pad), 1)
  ind = jnp.where(col == 0, jnp.float32(1.0), jnp.float32(0.0))
  h_ref[0, :, :half] = h[:, :half]
  h_ref[0, :, half:] = ind
  h_ref[1, :, :half] = h[:, half:]
  h_ref[1, :, half:] = ind


def _combine(acc_ref, b_ref, d_in):
  half = d_in // 2
  num = jnp.concatenate([acc_ref[0, :, :half], acc_ref[1, :, :half]], axis=1)
  den = acc_ref[0, :, half:half + 1]
  return jnp.maximum(num / (den + jnp.float32(1e-16)) + b_ref[...], 0.0)


def _dense_first_body(x_ref, w_ref, as_ref, ad_ref, h_ref, asrc_ref, adst_ref,
                      *, d_out, dp_out):
  h = jnp.dot(x_ref[...], w_ref[...], preferred_element_type=jnp.float32)
  _write_halves(h_ref, h, d_out, dp_out)
  asrc_ref[...] = jnp.dot(h, as_ref[...], preferred_element_type=jnp.float32)
  adst_ref[...] = jnp.dot(h, ad_ref[...], preferred_element_type=jnp.float32)


def _dense_mid_body(acc_ref, b_ref, w_ref, as_ref, ad_ref,
                    h_ref, asrc_ref, adst_ref, *, d_in, d_out, dp_out):
  hprev = _combine(acc_ref, b_ref, d_in)
  h = jnp.dot(hprev, w_ref[...], preferred_element_type=jnp.float32)
  _write_halves(h_ref, h, d_out, dp_out)
  asrc_ref[...] = jnp.dot(h, as_ref[...], preferred_element_type=jnp.float32)
  adst_ref[...] = jnp.dot(h, ad_ref[...], preferred_element_type=jnp.float32)


def _final_body(acc_ref, b_ref, o_ref, *, d):
  h = _combine(acc_ref, b_ref, d)
  m = jnp.max(h, axis=-1, keepdims=True)
  ex = jnp.exp(h - m)
  o_ref[...] = h - m - jnp.log(jnp.sum(ex, axis=-1, keepdims=True))


def _dense_first(x, w, a_s, a_d, d_out, dp_out):
  return pl.pallas_call(
      functools.partial(_dense_first_body, d_out=d_out, dp_out=dp_out),
      grid=(N // RB,),
      in_specs=[
          pl.BlockSpec((RB, x.shape[1]), lambda i: (i, 0)),
          pl.BlockSpec(w.shape, lambda i: (0, 0)),
          pl.BlockSpec((d_out, 1), lambda i: (0, 0)),
          pl.BlockSpec((d_out, 1), lambda i: (0, 0)),
      ],
      out_specs=[
          pl.BlockSpec((2, RB, dp_out), lambda i: (0, i, 0)),
          pl.BlockSpec((RB, 1), lambda i: (i, 0)),
          pl.BlockSpec((RB, 1), lambda i: (i, 0)),
      ],
      out_shape=[
          jax.ShapeDtypeStruct((2, N, dp_out), jnp.float32),
          jax.ShapeDtypeStruct((N, 1), jnp.float32),
          jax.ShapeDtypeStruct((N, 1), jnp.float32),
      ],
  )(x, w, a_s, a_d)


def _dense_mid(acc, b, w, a_s, a_d, d_in, d_out, dp_out):
  dp_in = acc.shape[2]
  return pl.pallas_call(
      functools.partial(_dense_mid_body, d_in=d_in, d_out=d_out,
                        dp_out=dp_out),
      grid=(N // RB,),
      in_specs=[
          pl.BlockSpec((2, RB, dp_in), lambda i: (0, i, 0)),
          pl.BlockSpec((1, d_in), lambda i: (0, 0)),
          pl.BlockSpec(w.shape, lambda i: (0, 0)),
          pl.BlockSpec((d_out, 1), lambda i: (0, 0)),
          pl.BlockSpec((d_out, 1), lambda i: (0, 0)),
      ],
      out_specs=[
          pl.BlockSpec((2, RB, dp_out), lambda i: (0, i, 0)),
          pl.BlockSpec((RB, 1), lambda i: (i, 0)),
          pl.BlockSpec((RB, 1), lambda i: (i, 0)),
      ],
      out_shape=[
          jax.ShapeDtypeStruct((2, N, dp_out), jnp.float32),
          jax.ShapeDtypeStruct((N, 1), jnp.float32),
          jax.ShapeDtypeStruct((N, 1), jnp.float32),
      ],
  )(acc, b, w, a_s, a_d)


def _final(acc, b, d):
  dp_in = acc.shape[2]
  return pl.pallas_call(
      functools.partial(_final_body, d=d),
      grid=(N // RB,),
      in_specs=[
          pl.BlockSpec((2, RB, dp_in), lambda i: (0, i, 0)),
          pl.BlockSpec((1, d), lambda i: (0, 0)),
      ],
      out_specs=pl.BlockSpec((RB, d), lambda i: (i, 0)),
      out_shape=jax.ShapeDtypeStruct((N, d), jnp.float32),
  )(acc, b)


def kernel(x, edge_index, edge_attr, W1, a_src1, a_dst1, b1,
           W2, a_src2, a_dst2, b2, W3, a_src3, a_dst3, b3):
  del edge_attr
  srcb = edge_index[0].reshape(16, NBT, B)
  dstb = edge_index[1].reshape(16, NBT, B)

  h1, as1, ad1 = _dense_first(x, W1, a_src1.reshape(-1, 1),
                              a_dst1.reshape(-1, 1), 128, 80)
  acc1 = _edge80(h1, as1.reshape(-1), ad1.reshape(-1), srcb, dstb)

  h2, as2, ad2 = _dense_mid(acc1, b1.reshape(1, -1), W2,
                            a_src2.reshape(-1, 1), a_dst2.reshape(-1, 1),
                            128, 128, 80)
  acc2 = _edge80(h2, as2.reshape(-1), ad2.reshape(-1), srcb, dstb)

  h3, as3, ad3 = _dense_mid(acc2, b2.reshape(1, -1), W3,
                            a_src3.reshape(-1, 1), a_dst3.reshape(-1, 1),
                            128, 64, 48)
  acc3 = _edge48(h3, as3.reshape(-1), ad3.reshape(-1), srcb, dstb)

  return _final(acc3, b3.reshape(1, -1), 64)


# speedup vs baseline: 34.6690x; 1.0008x over previous
import functools

import jax
import jax.numpy as jnp
from jax import lax
from jax.experimental import pallas as pl
from jax.experimental.pallas import tpu as pltpu
from jax.experimental.pallas import tpu_sc as plsc

N = 10000
E = 320000
B = 80
NB = E // B
NBT = NB // 16
TILE_ROWS = 624
TAIL_ROWS = N - 16 * TILE_ROWS
RB = 400


def _make_edge_kernel(dp):
  fg = dp // 16
  mesh = plsc.VectorSubcoreMesh(core_axis_name="c", subcore_axis_name="s")

  @functools.partial(
      pl.kernel,
      out_type=jax.ShapeDtypeStruct((2, N, dp), jnp.float32),
      mesh=mesh,
      compiler_params=pltpu.CompilerParams(
          needs_layout_passes=False, use_tc_tiling_on_sc=False,
          skip_device_barrier=True),
      scratch_types=[
          pltpu.VMEM((N,), jnp.float32),
          pltpu.VMEM((N,), jnp.float32),
          pltpu.VMEM((NBT, B), jnp.int32),
          pltpu.VMEM((NBT, B), jnp.int32),
          pltpu.VMEM((B,), jnp.float32),
          pltpu.VMEM((2, B, dp), jnp.float32),
          pltpu.VMEM_SHARED((N, dp), jnp.float32),
          pltpu.SemaphoreType.DMA,
          pltpu.SemaphoreType.DMA,
      ],
  )
  def edge_kernel(h_hbm, asrc_hbm, adst_hbm, srcb_hbm, dstb_hbm, out_hbm,
                  asrc_v, adst_v, srcb_v, dstb_v, ex_v, rows2_v,
                  acc_sh, gsem, ssem):
    cid = lax.axis_index("c")
    sid = lax.axis_index("s")

    pltpu.sync_copy(asrc_hbm, asrc_v)
    pltpu.sync_copy(adst_hbm, adst_v)
    pltpu.sync_copy(srcb_hbm.at[sid], srcb_v)
    pltpu.sync_copy(dstb_hbm.at[sid], dstb_v)

    zeros16 = jnp.zeros((16,), jnp.float32)
    zrows = rows2_v.at[0]

    def zrow(i, carry):
      for f in range(fg):
        zrows[i, pl.ds(f * 16, 16)] = zeros16
      return carry

    lax.fori_loop(0, B, zrow, 0)
    for kk in range(7):
      pltpu.sync_copy(zrows, acc_sh.at[pl.ds(sid * TILE_ROWS + kk * 80, 80)])
    pltpu.sync_copy(zrows.at[pl.ds(0, 64)],
                    acc_sh.at[pl.ds(sid * TILE_ROWS + 560, 64)])

    @pl.when(sid == 15)
    def _zero_tail():
      pltpu.sync_copy(zrows.at[pl.ds(0, TAIL_ROWS)],
                      acc_sh.at[pl.ds(16 * TILE_ROWS, TAIL_ROWS)])

    plsc.subcore_barrier()

    table = h_hbm.at[cid]

    pltpu.async_copy(table.at[srcb_v.at[0]], rows2_v.at[0], gsem)

    def do_batch(j, buf, other, first, last):
      rows_v = rows2_v.at[buf]

      @pl.when(jnp.logical_not(first))
      def _drain_scatter():
        pltpu.make_async_copy(
            rows2_v.at[other], acc_sh.at[pl.ds(0, B)], ssem).wait()

      @pl.when(jnp.logical_not(last))
      def _issue_gather():
        pltpu.async_copy(table.at[srcb_v.at[j + 1]], rows2_v.at[other], gsem)

      for g in range(B // 16):
        s16 = srcb_v[j, pl.ds(g * 16, 16)]
        d16 = dstb_v[j, pl.ds(g * 16, 16)]
        a_s = plsc.load_gather(asrc_v, [s16])
        a_d = plsc.load_gather(adst_v, [d16])
        e = a_s + a_d
        e = jnp.where(e < 0.0, e * jnp.float32(0.2), e)
        ex_v[pl.ds(g * 16, 16)] = jnp.exp(e)

      pltpu.make_async_copy(table.at[srcb_v.at[j]], rows_v, gsem).wait()

      for g in range(B // 16):
        ex16 = ex_v[pl.ds(g * 16, 16)]
        for k16 in range(16):
          exk = ex16[k16]
          row = g * 16 + k16
          for f in range(fg):
            rows_v[row, pl.ds(f * 16, 16)] = (
                rows_v[row, pl.ds(f * 16, 16)] * exk)
      pltpu.async_copy(rows_v, acc_sh.at[dstb_v.at[j]], ssem, add=True)

    def batch_pair(j2, carry):
      do_batch(2 * j2, 0, 1, first=j2 == 0, last=jnp.bool_(False))
      do_batch(2 * j2 + 1, 1, 0, first=jnp.bool_(False),
               last=j2 == NBT // 2 - 1)
      return carry

    lax.fori_loop(0, NBT // 2, batch_pair, 0)
    pltpu.make_async_copy(rows2_v.at[1], acc_sh.at[pl.ds(0, B)], ssem).wait()
    plsc.subcore_barrier()

    pltpu.sync_copy(acc_sh.at[pl.ds(sid * TILE_ROWS, TILE_ROWS)],
                    out_hbm.at[cid, pl.ds(sid * TILE_ROWS, TILE_ROWS)])

    @pl.when(sid == 15)
    def _copy_tail():
      pltpu.sync_copy(acc_sh.at[pl.ds(16 * TILE_ROWS, TAIL_ROWS)],
                      out_hbm.at[cid, pl.ds(16 * TILE_ROWS, TAIL_ROWS)])

  return edge_kernel


_edge80 = _make_edge_kernel(80)
_edge48 = _make_edge_kernel(48)


def _write_halves(h_ref, h, d_out, dp_out):
  half = d_out // 2
  pad = dp_out - half
  col = lax.broadcasted_iota(jnp.int32, (RB, pad), 1)
  ind = jnp.where(col == 0, jnp.float32(1.0), jnp.float32(0.0))
  h_ref[0, :, :half] = h[:, :half]
  h_ref[0, :, half:] = ind
  h_ref[1, :, :half] = h[:, half:]
  h_ref[1, :, half:] = ind


def _combine(acc_ref, b_ref, d_in):
  half = d_in // 2
  num = jnp.concatenate([acc_ref[0, :, :half], acc_ref[1, :, :half]], axis=1)
  den = acc_ref[0, :, half:half + 1]
  return jnp.maximum(num / (den + jnp.float32(1e-16)) + b_ref[...], 0.0)


def _dense_first_body(x_ref, w_ref, as_ref, ad_ref, h_ref, asrc_ref, adst_ref,
                      *, d_out, dp_out):
  h = jnp.dot(x_ref[...], w_ref[...], preferred_element_type=jnp.float32)
  _write_halves(h_ref, h, d_out, dp_out)
  asrc_ref[...] = jnp.dot(h, as_ref[...], preferred_element_type=jnp.float32)
  adst_ref[...] = jnp.dot(h, ad_ref[...], preferred_element_type=jnp.float32)


def _dense_mid_body(acc_ref, b_ref, w_ref, as_ref, ad_ref,
                    h_ref, asrc_ref, adst_ref, *, d_in, d_out, dp_out):
  hprev = _combine(acc_ref, b_ref, d_in)
  h = jnp.dot(hprev, w_ref[...], preferred_element_type=jnp.float32)
  _write_halves(h_ref, h, d_out, dp_out)
  asrc_ref[...] = jnp.dot(h, as_ref[...], preferred_element_type=jnp.float32)
  adst_ref[...] = jnp.dot(h, ad_ref[...], preferred_element_type=jnp.float32)


def _final_body(acc_ref, b_ref, o_ref, *, d):
  h = _combine(acc_ref, b_ref, d)
  m = jnp.max(h, axis=-1, keepdims=True)
  ex = jnp.exp(h - m)
  o_ref[...] = h - m - jnp.log(jnp.sum(ex, axis=-1, keepdims=True))


def _dense_first(x, w, a_s, a_d, d_out, dp_out):
  return pl.pallas_call(
      functools.partial(_dense_first_body, d_out=d_out, dp_out=dp_out),
      grid=(N // RB,),
      in_specs=[
          pl.BlockSpec((RB, x.shape[1]), lambda i: (i, 0)),
          pl.BlockSpec(w.shape, lambda i: (0, 0)),
          pl.BlockSpec((d_out, 1), lambda i: (0, 0)),
          pl.BlockSpec((d_out, 1), lambda i: (0, 0)),
      ],
      out_specs=[
          pl.BlockSpec((2, RB, dp_out), lambda i: (0, i, 0)),
          pl.BlockSpec((RB, 1), lambda i: (i, 0)),
          pl.BlockSpec((RB, 1), lambda i: (i, 0)),
      ],
      out_shape=[
          jax.ShapeDtypeStruct((2, N, dp_out), jnp.float32),
          jax.ShapeDtypeStruct((N, 1), jnp.float32),
          jax.ShapeDtypeStruct((N, 1), jnp.float32),
      ],
  )(x, w, a_s, a_d)


def _dense_mid(acc, b, w, a_s, a_d, d_in, d_out, dp_out):
  dp_in = acc.shape[2]
  return pl.pallas_call(
      functools.partial(_dense_mid_body, d_in=d_in, d_out=d_out,
                        dp_out=dp_out),
      grid=(N // RB,),
      in_specs=[
          pl.BlockSpec((2, RB, dp_in), lambda i: (0, i, 0)),
          pl.BlockSpec((1, d_in), lambda i: (0, 0)),
          pl.BlockSpec(w.shape, lambda i: (0, 0)),
          pl.BlockSpec((d_out, 1), lambda i: (0, 0)),
          pl.BlockSpec((d_out, 1), lambda i: (0, 0)),
      ],
      out_specs=[
          pl.BlockSpec((2, RB, dp_out), lambda i: (0, i, 0)),
          pl.BlockSpec((RB, 1), lambda i: (i, 0)),
          pl.BlockSpec((RB, 1), lambda i: (i, 0)),
      ],
      out_shape=[
          jax.ShapeDtypeStruct((2, N, dp_out), jnp.float32),
          jax.ShapeDtypeStruct((N, 1), jnp.float32),
          jax.ShapeDtypeStruct((N, 1), jnp.float32),
      ],
  )(acc, b, w, a_s, a_d)


def _final(acc, b, d):
  dp_in = acc.shape[2]
  return pl.pallas_call(
      functools.partial(_final_body, d=d),
      grid=(N // RB,),
      in_specs=[
          pl.BlockSpec((2, RB, dp_in), lambda i: (0, i, 0)),
          pl.BlockSpec((1, d), lambda i: (0, 0)),
      ],
      out_specs=pl.BlockSpec((RB, d), lambda i: (i, 0)),
      out_shape=jax.ShapeDtypeStruct((N, d), jnp.float32),
  )(acc, b)


def kernel(x, edge_index, edge_attr, W1, a_src1, a_dst1, b1,
           W2, a_src2, a_dst2, b2, W3, a_src3, a_dst3, b3):
  del edge_attr
  srcb = edge_index[0].reshape(16, NBT, B)
  dstb = edge_index[1].reshape(16, NBT, B)

  h1, as1, ad1 = _dense_first(x, W1, a_src1.reshape(-1, 1),
                              a_dst1.reshape(-1, 1), 128, 80)
  acc1 = _edge80(h1, as1.reshape(-1), ad1.reshape(-1), srcb, dstb)

  h2, as2, ad2 = _dense_mid(acc1, b1.reshape(1, -1), W2,
                            a_src2.reshape(-1, 1), a_dst2.reshape(-1, 1),
                            128, 128, 80)
  acc2 = _edge80(h2, as2.reshape(-1), ad2.reshape(-1), srcb, dstb)

  h3, as3, ad3 = _dense_mid(acc2, b2.reshape(1, -1), W3,
                            a_src3.reshape(-1, 1), a_dst3.reshape(-1, 1),
                            128, 64, 48)
  acc3 = _edge48(h3, as3.reshape(-1), ad3.reshape(-1), srcb, dstb)

  return _final(acc3, b3.reshape(1, -1), 64)


# speedup vs baseline: 46.0216x; 1.3275x over previous
import functools

import jax
import jax.numpy as jnp
from jax import lax
from jax.experimental import pallas as pl
from jax.experimental.pallas import tpu as pltpu
from jax.experimental.pallas import tpu_sc as plsc

N = 10000
E = 320000
B = 80
NB = E // B
NBT = NB // 16
TILE_ROWS = 624
TAIL_ROWS = N - 16 * TILE_ROWS


def _make_edge_kernel(dw):
  fg = dw // 16
  mesh = plsc.VectorSubcoreMesh(core_axis_name="c", subcore_axis_name="s")

  @functools.partial(
      pl.kernel,
      out_type=[
          jax.ShapeDtypeStruct((2, N, dw), jnp.float32),
          jax.ShapeDtypeStruct((2, 16, N), jnp.float32),
      ],
      mesh=mesh,
      compiler_params=pltpu.CompilerParams(
          needs_layout_passes=False, use_tc_tiling_on_sc=False,
          skip_device_barrier=True),
      scratch_types=[
          pltpu.VMEM((N,), jnp.float32),
          pltpu.VMEM((N,), jnp.float32),
          pltpu.VMEM((NBT, B), jnp.int32),
          pltpu.VMEM((NBT, B), jnp.int32),
          pltpu.VMEM((B,), jnp.float32),
          pltpu.VMEM((2, B, dw), jnp.float32),
          pltpu.VMEM((N,), jnp.float32),
          pltpu.VMEM_SHARED((N, dw), jnp.float32),
          pltpu.SemaphoreType.DMA,
          pltpu.SemaphoreType.DMA,
      ],
  )
  def edge_kernel(h_hbm, asrc_hbm, adst_hbm, srcb_hbm, dstb_hbm,
                  out_hbm, den_hbm,
                  asrc_v, adst_v, srcb_v, dstb_v, ex_v, rows2_v, den_v,
                  acc_sh, gsem, ssem):
    cid = lax.axis_index("c")
    sid = lax.axis_index("s")

    pltpu.sync_copy(asrc_hbm, asrc_v)
    pltpu.sync_copy(adst_hbm, adst_v)
    pltpu.sync_copy(srcb_hbm.at[sid], srcb_v)
    pltpu.sync_copy(dstb_hbm.at[sid], dstb_v)

    zeros16 = jnp.zeros((16,), jnp.float32)

    def zden(i, carry):
      den_v[pl.ds(i * 16, 16)] = zeros16
      return carry

    lax.fori_loop(0, N // 16, zden, 0)

    zrows = rows2_v.at[0]

    def zrow(i, carry):
      for f in range(fg):
        zrows[i, pl.ds(f * 16, 16)] = zeros16
      return carry

    lax.fori_loop(0, B, zrow, 0)
    for kk in range(7):
      pltpu.sync_copy(zrows, acc_sh.at[pl.ds(sid * TILE_ROWS + kk * 80, 80)])
    pltpu.sync_copy(zrows.at[pl.ds(0, 64)],
                    acc_sh.at[pl.ds(sid * TILE_ROWS + 560, 64)])

    @pl.when(sid == 15)
    def _zero_tail():
      pltpu.sync_copy(zrows.at[pl.ds(0, TAIL_ROWS)],
                      acc_sh.at[pl.ds(16 * TILE_ROWS, TAIL_ROWS)])

    plsc.subcore_barrier()

    table = h_hbm.at[cid]

    pltpu.async_copy(table.at[srcb_v.at[0]], rows2_v.at[0], gsem)

    def do_batch(j, buf, other, first, last):
      rows_v = rows2_v.at[buf]

      @pl.when(jnp.logical_not(first))
      def _drain_scatter():
        pltpu.make_async_copy(
            rows2_v.at[other], acc_sh.at[pl.ds(0, B)], ssem).wait()

      @pl.when(jnp.logical_not(last))
      def _issue_gather():
        pltpu.async_copy(table.at[srcb_v.at[j + 1]], rows2_v.at[other], gsem)

      for g in range(B // 16):
        s16 = srcb_v[j, pl.ds(g * 16, 16)]
        d16 = dstb_v[j, pl.ds(g * 16, 16)]
        a_s = plsc.load_gather(asrc_v, [s16])
        a_d = plsc.load_gather(adst_v, [d16])
        e = a_s + a_d
        e = jnp.where(e < 0.0, e * jnp.float32(0.2), e)
        ex16 = jnp.exp(e)
        ex_v[pl.ds(g * 16, 16)] = ex16
        mask = (jnp.zeros((16,), jnp.int32) + cid) == (g % 2)
        plsc.addupdate_scatter(den_v, [d16], ex16, mask=mask)

      pltpu.make_async_copy(table.at[srcb_v.at[j]], rows_v, gsem).wait()

      for g in range(B // 16):
        ex16 = ex_v[pl.ds(g * 16, 16)]
        for k16 in range(16):
          exk = ex16[k16]
          row = g * 16 + k16
          for f in range(fg):
            rows_v[row, pl.ds(f * 16, 16)] = (
                rows_v[row, pl.ds(f * 16, 16)] * exk)

      pltpu.async_copy(rows_v, acc_sh.at[dstb_v.at[j]], ssem, add=True)

    def batch_pair(j2, carry):
      do_batch(2 * j2, 0, 1, first=j2 == 0, last=jnp.bool_(False))
      do_batch(2 * j2 + 1, 1, 0, first=jnp.bool_(False),
               last=j2 == NBT // 2 - 1)
      return carry

    lax.fori_loop(0, NBT // 2, batch_pair, 0)
    pltpu.make_async_copy(rows2_v.at[1], acc_sh.at[pl.ds(0, B)], ssem).wait()

    pltpu.sync_copy(den_v, den_hbm.at[cid, sid])
    plsc.subcore_barrier()

    pltpu.sync_copy(acc_sh.at[pl.ds(sid * TILE_ROWS, TILE_ROWS)],
                    out_hbm.at[cid, pl.ds(sid * TILE_ROWS, TILE_ROWS)])

    @pl.when(sid == 15)
    def _copy_tail():
      pltpu.sync_copy(acc_sh.at[pl.ds(16 * TILE_ROWS, TAIL_ROWS)],
                      out_hbm.at[cid, pl.ds(16 * TILE_ROWS, TAIL_ROWS)])

  return edge_kernel


_edge64 = _make_edge_kernel(64)
_edge32 = _make_edge_kernel(32)


def _write_halves(h_ref, h, d_out):
  half = d_out // 2
  h_ref[0] = h[:, :half]
  h_ref[1] = h[:, half:]


def _combine(acc_ref, den_ref, b_ref):
  num = jnp.concatenate([acc_ref[0], acc_ref[1]], axis=1)
  den = jnp.sum(den_ref[...], axis=(0, 1)).reshape(N, 1)
  return jnp.maximum(num / (den + jnp.float32(1e-16)) + b_ref[...], 0.0)


def _dense_first_body(x_ref, w_ref, as_ref, ad_ref, h_ref, asrc_ref, adst_ref,
                      *, d_out):
  h = jnp.dot(x_ref[...], w_ref[...], preferred_element_type=jnp.float32)
  _write_halves(h_ref, h, d_out)
  asrc_ref[...] = jnp.dot(h, as_ref[...], preferred_element_type=jnp.float32)
  adst_ref[...] = jnp.dot(h, ad_ref[...], preferred_element_type=jnp.float32)


def _dense_mid_body(acc_ref, den_ref, b_ref, w_ref, as_ref, ad_ref,
                    h_ref, asrc_ref, adst_ref, *, d_out):
  hprev = _combine(acc_ref, den_ref, b_ref)
  h = jnp.dot(hprev, w_ref[...], preferred_element_type=jnp.float32)
  _write_halves(h_ref, h, d_out)
  asrc_ref[...] = jnp.dot(h, as_ref[...], preferred_element_type=jnp.float32)
  adst_ref[...] = jnp.dot(h, ad_ref[...], preferred_element_type=jnp.float32)


def _final_body(acc_ref, den_ref, b_ref, o_ref):
  h = _combine(acc_ref, den_ref, b_ref)
  m = jnp.max(h, axis=-1, keepdims=True)
  ex = jnp.exp(h - m)
  o_ref[...] = h - m - jnp.log(jnp.sum(ex, axis=-1, keepdims=True))


def _dense_first(x, w, a_s, a_d, d_out):
  return pl.pallas_call(
      functools.partial(_dense_first_body, d_out=d_out),
      out_shape=[
          jax.ShapeDtypeStruct((2, N, d_out // 2), jnp.float32),
          jax.ShapeDtypeStruct((N, 1), jnp.float32),
          jax.ShapeDtypeStruct((N, 1), jnp.float32),
      ],
  )(x, w, a_s, a_d)


def _dense_mid(acc, den, b, w, a_s, a_d, d_out):
  return pl.pallas_call(
      functools.partial(_dense_mid_body, d_out=d_out),
      out_shape=[
          jax.ShapeDtypeStruct((2, N, d_out // 2), jnp.float32),
          jax.ShapeDtypeStruct((N, 1), jnp.float32),
          jax.ShapeDtypeStruct((N, 1), jnp.float32),
      ],
  )(acc, den, b, w, a_s, a_d)


def _final(acc, den, b, d):
  return pl.pallas_call(
      _final_body,
      out_shape=jax.ShapeDtypeStruct((N, d), jnp.float32),
  )(acc, den, b)


def kernel(x, edge_index, edge_attr, W1, a_src1, a_dst1, b1,
           W2, a_src2, a_dst2, b2, W3, a_src3, a_dst3, b3):
  del edge_attr
  srcb = edge_index[0].reshape(16, NBT, B)
  dstb = edge_index[1].reshape(16, NBT, B)

  h1, as1, ad1 = _dense_first(x, W1, a_src1.reshape(-1, 1),
                              a_dst1.reshape(-1, 1), 128)
  acc1, den1 = _edge64(h1, as1.reshape(-1), ad1.reshape(-1), srcb, dstb)

  h2, as2, ad2 = _dense_mid(acc1, den1, b1.reshape(1, -1), W2,
                            a_src2.reshape(-1, 1), a_dst2.reshape(-1, 1), 128)
  acc2, den2 = _edge64(h2, as2.reshape(-1), ad2.reshape(-1), srcb, dstb)

  h3, as3, ad3 = _dense_mid(acc2, den2, b2.reshape(1, -1), W3,
                            a_src3.reshape(-1, 1), a_dst3.reshape(-1, 1), 64)
  acc3, den3 = _edge32(h3, as3.reshape(-1), ad3.reshape(-1), srcb, dstb)

  return _final(acc3, den3, b3.reshape(1, -1), 64)


# speedup vs baseline: 47.0999x; 1.0234x over previous
import functools

import jax
import jax.numpy as jnp
from jax import lax
from jax.experimental import pallas as pl
from jax.experimental.pallas import tpu as pltpu
from jax.experimental.pallas import tpu_sc as plsc

N = 10000
E = 320000
B = 80
NB = E // B
NBT = NB // 16
TILE_ROWS = 624
TAIL_ROWS = N - 16 * TILE_ROWS


def _make_edge_kernel(dw, bf16_table=False):
  fg = dw // 16
  mesh = plsc.VectorSubcoreMesh(core_axis_name="c", subcore_axis_name="s")
  table_dtype = jnp.bfloat16 if bf16_table else jnp.float32

  @functools.partial(
      pl.kernel,
      out_type=[
          jax.ShapeDtypeStruct((2, N, dw), jnp.float32),
          jax.ShapeDtypeStruct((2, 16, N), jnp.float32),
      ],
      mesh=mesh,
      compiler_params=pltpu.CompilerParams(
          needs_layout_passes=False, use_tc_tiling_on_sc=False,
          skip_device_barrier=True),
      scratch_types=[
          pltpu.VMEM((N,), jnp.float32),
          pltpu.VMEM((N,), jnp.float32),
          pltpu.VMEM((NBT, B), jnp.int32),
          pltpu.VMEM((NBT, B), jnp.int32),
          pltpu.VMEM((B,), jnp.float32),
          pltpu.VMEM((2, B, dw), jnp.float32),
          pltpu.VMEM((2, B, dw), table_dtype),
          pltpu.VMEM((N,), jnp.float32),
          pltpu.VMEM_SHARED((N, dw), jnp.float32),
          pltpu.SemaphoreType.DMA,
          pltpu.SemaphoreType.DMA,
      ],
  )
  def edge_kernel(h_hbm, asrc_hbm, adst_hbm, srcb_hbm, dstb_hbm,
                  out_hbm, den_hbm,
                  asrc_v, adst_v, srcb_v, dstb_v, ex_v, rows2_v, gb2_v, den_v,
                  acc_sh, gsem, ssem):
    cid = lax.axis_index("c")
    sid = lax.axis_index("s")

    pltpu.sync_copy(asrc_hbm, asrc_v)
    pltpu.sync_copy(adst_hbm, adst_v)
    pltpu.sync_copy(srcb_hbm.at[sid], srcb_v)
    pltpu.sync_copy(dstb_hbm.at[sid], dstb_v)

    zeros16 = jnp.zeros((16,), jnp.float32)

    def zden(i, carry):
      den_v[pl.ds(i * 16, 16)] = zeros16
      return carry

    lax.fori_loop(0, N // 16, zden, 0)

    zrows = rows2_v.at[0]

    def zrow(i, carry):
      for f in range(fg):
        zrows[i, pl.ds(f * 16, 16)] = zeros16
      return carry

    lax.fori_loop(0, B, zrow, 0)
    for kk in range(7):
      pltpu.sync_copy(zrows, acc_sh.at[pl.ds(sid * TILE_ROWS + kk * 80, 80)])
    pltpu.sync_copy(zrows.at[pl.ds(0, 64)],
                    acc_sh.at[pl.ds(sid * TILE_ROWS + 560, 64)])

    @pl.when(sid == 15)
    def _zero_tail():
      pltpu.sync_copy(zrows.at[pl.ds(0, TAIL_ROWS)],
                      acc_sh.at[pl.ds(16 * TILE_ROWS, TAIL_ROWS)])

    plsc.subcore_barrier()

    table = h_hbm.at[cid]

    pltpu.async_copy(table.at[srcb_v.at[0]], gb2_v.at[0], gsem)

    def do_batch(j, buf, other, first, last):
      rows_v = rows2_v.at[buf]
      gb_v = gb2_v.at[buf]

      @pl.when(jnp.logical_not(first))
      def _drain_scatter():
        pltpu.make_async_copy(
            rows2_v.at[other], acc_sh.at[pl.ds(0, B)], ssem).wait()

      @pl.when(jnp.logical_not(last))
      def _issue_gather():
        pltpu.async_copy(table.at[srcb_v.at[j + 1]], gb2_v.at[other], gsem)

      for g in range(B // 16):
        s16 = srcb_v[j, pl.ds(g * 16, 16)]
        d16 = dstb_v[j, pl.ds(g * 16, 16)]
        a_s = plsc.load_gather(asrc_v, [s16])
        a_d = plsc.load_gather(adst_v, [d16])
        e = a_s + a_d
        e = jnp.where(e < 0.0, e * jnp.float32(0.2), e)
        ex16 = jnp.exp(e)
        ex_v[pl.ds(g * 16, 16)] = ex16
        mask = (jnp.zeros((16,), jnp.int32) + cid) == (g % 2)
        plsc.addupdate_scatter(den_v, [d16], ex16, mask=mask)

      pltpu.make_async_copy(table.at[srcb_v.at[j]], gb_v, gsem).wait()

      for g in range(B // 16):
        ex16 = ex_v[pl.ds(g * 16, 16)]
        for k16 in range(16):
          exk = ex16[k16]
          row = g * 16 + k16
          if bf16_table:
            for c in range(dw // 32):
              packed = plsc.bitcast(gb_v[row, pl.ds(c * 32, 32)], jnp.int32)
              ev = plsc.bitcast(packed << 16, jnp.float32)
              od = plsc.bitcast(packed & jnp.int32(-65536), jnp.float32)
              rows_v[row, pl.ds(c * 32, 16)] = ev * exk
              rows_v[row, pl.ds(c * 32 + 16, 16)] = od * exk
          else:
            for f in range(fg):
              rows_v[row, pl.ds(f * 16, 16)] = (
                  gb_v[row, pl.ds(f * 16, 16)] * exk)

      pltpu.async_copy(rows_v, acc_sh.at[dstb_v.at[j]], ssem, add=True)

    def batch_pair(j2, carry):
      do_batch(2 * j2, 0, 1, first=j2 == 0, last=jnp.bool_(False))
      do_batch(2 * j2 + 1, 1, 0, first=jnp.bool_(False),
               last=j2 == NBT // 2 - 1)
      return carry

    lax.fori_loop(0, NBT // 2, batch_pair, 0)
    pltpu.make_async_copy(rows2_v.at[1], acc_sh.at[pl.ds(0, B)], ssem).wait()

    pltpu.sync_copy(den_v, den_hbm.at[cid, sid])
    plsc.subcore_barrier()

    pltpu.sync_copy(acc_sh.at[pl.ds(sid * TILE_ROWS, TILE_ROWS)],
                    out_hbm.at[cid, pl.ds(sid * TILE_ROWS, TILE_ROWS)])

    @pl.when(sid == 15)
    def _copy_tail():
      pltpu.sync_copy(acc_sh.at[pl.ds(16 * TILE_ROWS, TAIL_ROWS)],
                      out_hbm.at[cid, pl.ds(16 * TILE_ROWS, TAIL_ROWS)])

  return edge_kernel


_edge64 = _make_edge_kernel(64, bf16_table=True)
_edge32 = _make_edge_kernel(32)

import numpy as _np

_ph = _np.arange(64)
_PH = 32 * (_ph // 32) + 2 * (_ph % 16) + ((_ph % 32) // 16)
_PERM128 = _np.concatenate([_PH, 64 + _PH])


def _write_halves(h_ref, h, d_out):
  half = d_out // 2
  h_ref[0] = h[:, :half].astype(h_ref.dtype)
  h_ref[1] = h[:, half:].astype(h_ref.dtype)


def _combine(acc_ref, den_ref, b_ref):
  num = jnp.concatenate([acc_ref[0], acc_ref[1]], axis=1)
  den = jnp.sum(den_ref[...], axis=(0, 1)).reshape(N, 1)
  return jnp.maximum(num / (den + jnp.float32(1e-16)) + b_ref[...], 0.0)


def _dense_first_body(x_ref, w_ref, as_ref, ad_ref, h_ref, asrc_ref, adst_ref,
                      *, d_out):
  h = jnp.dot(x_ref[...], w_ref[...], preferred_element_type=jnp.float32)
  _write_halves(h_ref, h, d_out)
  asrc_ref[...] = jnp.dot(h, as_ref[...], preferred_element_type=jnp.float32)
  adst_ref[...] = jnp.dot(h, ad_ref[...], preferred_element_type=jnp.float32)


def _dense_mid_body(acc_ref, den_ref, b_ref, w_ref, as_ref, ad_ref,
                    h_ref, asrc_ref, adst_ref, *, d_out):
  hprev = _combine(acc_ref, den_ref, b_ref)
  h = jnp.dot(hprev, w_ref[...], preferred_element_type=jnp.float32)
  _write_halves(h_ref, h, d_out)
  asrc_ref[...] = jnp.dot(h, as_ref[...], preferred_element_type=jnp.float32)
  adst_ref[...] = jnp.dot(h, ad_ref[...], preferred_element_type=jnp.float32)


def _final_body(acc_ref, den_ref, b_ref, o_ref):
  h = _combine(acc_ref, den_ref, b_ref)
  m = jnp.max(h, axis=-1, keepdims=True)
  ex = jnp.exp(h - m)
  o_ref[...] = h - m - jnp.log(jnp.sum(ex, axis=-1, keepdims=True))


def _dense_first(x, w, a_s, a_d, d_out, table_dtype):
  return pl.pallas_call(
      functools.partial(_dense_first_body, d_out=d_out),
      out_shape=[
          jax.ShapeDtypeStruct((2, N, d_out // 2), table_dtype),
          jax.ShapeDtypeStruct((N, 1), jnp.float32),
          jax.ShapeDtypeStruct((N, 1), jnp.float32),
      ],
  )(x, w, a_s, a_d)


def _dense_mid(acc, den, b, w, a_s, a_d, d_out, table_dtype):
  return pl.pallas_call(
      functools.partial(_dense_mid_body, d_out=d_out),
      out_shape=[
          jax.ShapeDtypeStruct((2, N, d_out // 2), table_dtype),
          jax.ShapeDtypeStruct((N, 1), jnp.float32),
          jax.ShapeDtypeStruct((N, 1), jnp.float32),
      ],
  )(acc, den, b, w, a_s, a_d)


def _final(acc, den, b, d):
  return pl.pallas_call(
      _final_body,
      out_shape=jax.ShapeDtypeStruct((N, d), jnp.float32),
  )(acc, den, b)


def kernel(x, edge_index, edge_attr, W1, a_src1, a_dst1, b1,
           W2, a_src2, a_dst2, b2, W3, a_src3, a_dst3, b3):
  del edge_attr
  srcb = edge_index[0].reshape(16, NBT, B)
  dstb = edge_index[1].reshape(16, NBT, B)

  h1, as1, ad1 = _dense_first(x, W1, a_src1.reshape(-1, 1),
                              a_dst1.reshape(-1, 1), 128, jnp.bfloat16)
  acc1, den1 = _edge64(h1, as1.reshape(-1), ad1.reshape(-1), srcb, dstb)

  h2, as2, ad2 = _dense_mid(acc1, den1, b1[_PERM128].reshape(1, -1),
                            W2[_PERM128], a_src2.reshape(-1, 1),
                            a_dst2.reshape(-1, 1), 128, jnp.bfloat16)
  acc2, den2 = _edge64(h2, as2.reshape(-1), ad2.reshape(-1), srcb, dstb)

  h3, as3, ad3 = _dense_mid(acc2, den2, b2[_PERM128].reshape(1, -1),
                            W3[_PERM128], a_src3.reshape(-1, 1),
                            a_dst3.reshape(-1, 1), 64, jnp.float32)
  acc3, den3 = _edge32(h3, as3.reshape(-1), ad3.reshape(-1), srcb, dstb)

  return _final(acc3, den3, b3.reshape(1, -1), 64)
